# Initial kernel scaffold; baseline (speedup 1.0000x reference)
import jax
import jax.numpy as jnp
from jax.experimental import pallas as pl


def kernel(queries, keys, values, attn_mask):
    raise NotImplementedError("write your pallas kernel here")


# speedup vs baseline: 2.4535x; 2.4535x over previous
import math

import jax
import jax.numpy as jnp
from jax.experimental import pallas as pl
from jax.experimental.pallas import tpu as pltpu


_TILE_R = 256


def _corr_kernel(q_ref, kr_ref, s_ref):
    _, L, HE = kr_ref.shape
    R = _TILE_R
    a = q_ref[0]
    g = jax.lax.dot_general(a, kr_ref[0], (((1,), (1,)), ((), ())),
                            preferred_element_type=jnp.float32)
    rows = jax.lax.broadcasted_iota(jnp.int32, (R, L), 0)
    for bit in range(R.bit_length() - 1):
        sh = 1 << bit
        rolled = jnp.roll(g, sh, axis=1)
        g = jnp.where((rows >> bit) & 1 == 1, rolled, g)
    s_ref[0, 0] = jnp.sum(g, axis=0, keepdims=True) * (1.0 / HE)


def _topk_kernel(s_ref, idx_ref, w_ref, topk: int):
    Bsz, J, _, L = s_ref.shape
    corr = jnp.zeros((Bsz, L), jnp.float32)
    for j in range(J):
        t0 = j * _TILE_R
        sj = s_ref[:, j, 0, :]
        corr = corr + (sj if t0 == 0 else jnp.roll(sj, t0, axis=1))
    score = jnp.mean(corr, axis=0, keepdims=True)
    lane = jax.lax.broadcasted_iota(jnp.int32, (1, L), 1)
    cols = []
    for i in range(topk):
        m = jnp.max(score)
        idx_i = jnp.min(jnp.where(score == m, lane, L))
        idx_ref[i] = idx_i
        cols.append(jnp.sum(jnp.where(lane == idx_i, corr, 0.0), axis=1,
                            keepdims=True))
        score = jnp.where(lane == idx_i, -jnp.inf, score)
    w = jnp.concatenate(cols, axis=1)
    m = jnp.max(w, axis=1, keepdims=True)
    e = jnp.exp(w - m)
    w = e / jnp.sum(e, axis=1, keepdims=True)
    w_ref[...] = jnp.concatenate(
        [w, jnp.zeros((Bsz, 8 - topk), jnp.float32)], axis=1)


def _agg_kernel(idx_ref, w_ref, v_hbm, out_ref, buf, sems, topk: int,
                blk_l: int):
    b = pl.program_id(0)
    j = pl.program_id(1)
    l0 = j * blk_l
    n = blk_l + 8
    copies = []
    for i in range(topk):
        base = l0 + idx_ref[i]
        off = jax.lax.rem(base, 8)
        aligned = pl.multiple_of(base - off, 8)
        c = pltpu.make_async_copy(v_hbm.at[b, pl.ds(aligned, n), :],
                                  buf.at[i], sems.at[i])
        c.start()
        copies.append((c, off))
    acc = None
    for i, (c, off) in enumerate(copies):
        c.wait()
        wv = w_ref[b, i]
        shift = jnp.where(off == 0, 0, n - off)
        rolled = pltpu.roll(buf[i], shift, axis=0)
        term = rolled[0:blk_l, :] * wv
        acc = term if acc is None else acc + term
    out_ref[0] = acc


def kernel(queries, keys, values, attn_mask):
    B, L, H, E = queries.shape
    HE = H * E
    topk = int(math.log(L))
    q2 = queries.reshape(B, L, HE)
    k2 = keys.reshape(B, L, HE)
    k_rev = jnp.roll(jnp.flip(k2, axis=1), 1, axis=1)
    v2 = values.reshape(B, L, HE)
    v_ext = jnp.concatenate([v2, v2], axis=1)

    J = L // _TILE_R
    s_tiles = pl.pallas_call(
        _corr_kernel,
        grid=(B, J),
        in_specs=[
            pl.BlockSpec((1, _TILE_R, HE), lambda b, j: (b, j, 0)),
            pl.BlockSpec((1, L, HE), lambda b, j: (b, 0, 0)),
        ],
        out_specs=pl.BlockSpec((1, 1, 1, L), lambda b, j: (b, j, 0, 0)),
        out_shape=jax.ShapeDtypeStruct((B, J, 1, L), jnp.float32),
    )(q2, k_rev)

    idx, w = pl.pallas_call(
        lambda c, i, wo: _topk_kernel(c, i, wo, topk),
        in_specs=[pl.BlockSpec((B, J, 1, L), lambda: (0, 0, 0, 0))],
        out_specs=[
            pl.BlockSpec(memory_space=pltpu.SMEM),
            pl.BlockSpec((B, 8), lambda: (0, 0)),
        ],
        out_shape=[
            jax.ShapeDtypeStruct((8,), jnp.int32),
            jax.ShapeDtypeStruct((B, 8), jnp.float32),
        ],
    )(s_tiles)

    blk_l = 256
    out = pl.pallas_call(
        lambda i, wi, v, o, buf, sems: _agg_kernel(i, wi, v, o, buf, sems,
                                                   topk, blk_l),
        grid=(B, L // blk_l),
        in_specs=[
            pl.BlockSpec(memory_space=pltpu.SMEM),
            pl.BlockSpec(memory_space=pltpu.SMEM),
            pl.BlockSpec(memory_space=pl.ANY),
        ],
        out_specs=pl.BlockSpec((1, blk_l, HE), lambda b, j: (b, j, 0)),
        out_shape=jax.ShapeDtypeStruct((B, L, HE), jnp.float32),
        scratch_shapes=[
            pltpu.VMEM((topk, blk_l + 8, HE), jnp.float32),
            pltpu.SemaphoreType.DMA((topk,)),
        ],
    )(idx, w, v_ext)

    return out.reshape(B, L, H, E)


# speedup vs baseline: 3.0571x; 1.2460x over previous
import math

import jax
import jax.numpy as jnp
from jax.experimental import pallas as pl
from jax.experimental.pallas import tpu as pltpu


_TILE_R = 256


def _corr_kernel(q_ref, k_ref, s_ref):
    _, L, HE = k_ref.shape
    R = _TILE_R
    g = jax.lax.dot_general(q_ref[0], k_ref[0], (((1,), (1,)), ((), ())),
                            precision=jax.lax.Precision.HIGHEST,
                            preferred_element_type=jnp.float32)
    rows = jax.lax.broadcasted_iota(jnp.int32, (R, L), 0)
    for bit in range(R.bit_length() - 1):
        sh = 1 << bit
        rolled = jnp.roll(g, -sh, axis=1)
        g = jnp.where((rows >> bit) & 1 == 1, rolled, g)
    s_ref[0, 0] = jnp.sum(g, axis=0, keepdims=True) * (1.0 / HE)


def _topk_kernel(s_ref, idx_ref, w_ref, topk: int):
    Bsz, J, _, L = s_ref.shape
    u = jnp.zeros((Bsz, L), jnp.float32)
    for j in range(J):
        t0 = j * _TILE_R
        sj = s_ref[:, j, 0, :]
        u = u + (sj if t0 == 0 else jnp.roll(sj, -t0, axis=1))
    score = jnp.mean(u, axis=0, keepdims=True)
    lane = jax.lax.broadcasted_iota(jnp.int32, (1, L), 1)
    cols = []
    for i in range(topk):
        m = jnp.max(score)
        c_i = jnp.min(jnp.where(score == m, lane, L))
        idx_ref[i] = jnp.where(c_i == 0, 0, L - c_i)
        cols.append(jnp.sum(jnp.where(lane == c_i, u, 0.0), axis=1,
                            keepdims=True))
        score = jnp.where(lane == c_i, -jnp.inf, score)
    w = jnp.concatenate(cols, axis=1)
    m = jnp.max(w, axis=1, keepdims=True)
    e = jnp.exp(w - m)
    w = e / jnp.sum(e, axis=1, keepdims=True)
    w_ref[...] = jnp.concatenate(
        [w, jnp.zeros((Bsz, 8 - topk), jnp.float32)], axis=1)


def _agg_kernel(idx_ref, w_ref, v_hbm, out_ref, buf, sems, topk: int,
                blk_l: int, L: int):
    b = pl.program_id(0)
    j = pl.program_id(1)
    l0 = j * blk_l
    n = blk_l + 8
    copies = []
    for i in range(topk):
        base = jax.lax.rem(l0 + idx_ref[i], L)
        off = jax.lax.rem(base, 8)
        aligned = pl.multiple_of(base - off, 8)
        c = pltpu.make_async_copy(v_hbm.at[b, pl.ds(aligned, n), :],
                                  buf.at[i], sems.at[i])
        c.start()
        copies.append((c, off))
    acc = None
    for i, (c, off) in enumerate(copies):
        c.wait()
        wv = w_ref[b, i]
        shift = jnp.where(off == 0, 0, n - off)
        rolled = pltpu.roll(buf[i], shift, axis=0)
        term = rolled[0:blk_l, :] * wv
        acc = term if acc is None else acc + term
    out_ref[0] = acc


def kernel(queries, keys, values, attn_mask):
    B, L, H, E = queries.shape
    HE = H * E
    topk = int(math.log(L))
    blk_l = 256
    q2 = queries.reshape(B, L, HE)
    k2 = keys.reshape(B, L, HE)
    v2 = values.reshape(B, L, HE)
    v_pad = jnp.concatenate([v2, v2[:, :blk_l + 8]], axis=1)

    J = L // _TILE_R
    s_tiles = pl.pallas_call(
        _corr_kernel,
        grid=(B, J),
        in_specs=[
            pl.BlockSpec((1, _TILE_R, HE), lambda b, j: (b, j, 0)),
            pl.BlockSpec((1, L, HE), lambda b, j: (b, 0, 0)),
        ],
        out_specs=pl.BlockSpec((1, 1, 1, L), lambda b, j: (b, j, 0, 0)),
        out_shape=jax.ShapeDtypeStruct((B, J, 1, L), jnp.float32),
    )(q2, k2)

    idx, w = pl.pallas_call(
        lambda c, i, wo: _topk_kernel(c, i, wo, topk),
        in_specs=[pl.BlockSpec((B, J, 1, L), lambda: (0, 0, 0, 0))],
        out_specs=[
            pl.BlockSpec(memory_space=pltpu.SMEM),
            pl.BlockSpec((B, 8), lambda: (0, 0)),
        ],
        out_shape=[
            jax.ShapeDtypeStruct((8,), jnp.int32),
            jax.ShapeDtypeStruct((B, 8), jnp.float32),
        ],
    )(s_tiles)

    out = pl.pallas_call(
        lambda i, wi, v, o, buf, sems: _agg_kernel(i, wi, v, o, buf, sems,
                                                   topk, blk_l, L),
        grid=(B, L // blk_l),
        in_specs=[
            pl.BlockSpec(memory_space=pltpu.SMEM),
            pl.BlockSpec(memory_space=pltpu.SMEM),
            pl.BlockSpec(memory_space=pl.ANY),
        ],
        out_specs=pl.BlockSpec((1, blk_l, HE), lambda b, j: (b, j, 0)),
        out_shape=jax.ShapeDtypeStruct((B, L, HE), jnp.float32),
        scratch_shapes=[
            pltpu.VMEM((topk, blk_l + 8, HE), jnp.float32),
            pltpu.SemaphoreType.DMA((topk,)),
        ],
    )(idx, w, v_pad)

    return out.reshape(B, L, H, E)


# speedup vs baseline: 3.4635x; 1.1329x over previous
import math

import jax
import jax.numpy as jnp
from jax.experimental import pallas as pl
from jax.experimental.pallas import tpu as pltpu


_TILE_R = 256


def _corr_kernel(q_ref, k_ref, s_ref):
    _, L, HE = k_ref.shape
    R = _TILE_R
    a = q_ref[0]
    kk = k_ref[0]
    a_hi = a.astype(jnp.bfloat16)
    a_lo = (a - a_hi.astype(jnp.float32)).astype(jnp.bfloat16)
    k_hi = kk.astype(jnp.bfloat16)
    k_lo = (kk - k_hi.astype(jnp.float32)).astype(jnp.bfloat16)
    dot = lambda x, y: jax.lax.dot_general(
        x, y, (((1,), (1,)), ((), ())), preferred_element_type=jnp.float32)
    g = dot(a_hi, k_hi) + dot(a_hi, k_lo) + dot(a_lo, k_hi)
    rows = jax.lax.broadcasted_iota(jnp.int32, (R, L), 0)
    for bit in range(R.bit_length() - 1):
        sh = 1 << bit
        rolled = jnp.roll(g, -sh, axis=1)
        g = jnp.where((rows >> bit) & 1 == 1, rolled, g)
    s_ref[0, 0] = jnp.sum(g, axis=0, keepdims=True) * (1.0 / HE)


def _topk_kernel(s_ref, idx_ref, w_ref, topk: int):
    Bsz, J, _, L = s_ref.shape
    u = jnp.zeros((Bsz, L), jnp.float32)
    for j in range(J):
        t0 = j * _TILE_R
        sj = s_ref[:, j, 0, :]
        u = u + (sj if t0 == 0 else jnp.roll(sj, -t0, axis=1))
    score = jnp.mean(u, axis=0, keepdims=True)
    lane = jax.lax.broadcasted_iota(jnp.int32, (1, L), 1)
    cols = []
    for i in range(topk):
        m = jnp.max(score)
        c_i = jnp.min(jnp.where(score == m, lane, L))
        idx_ref[i] = jnp.where(c_i == 0, 0, L - c_i)
        cols.append(jnp.sum(jnp.where(lane == c_i, u, 0.0), axis=1,
                            keepdims=True))
        score = jnp.where(lane == c_i, -jnp.inf, score)
    w = jnp.concatenate(cols, axis=1)
    m = jnp.max(w, axis=1, keepdims=True)
    e = jnp.exp(w - m)
    w = e / jnp.sum(e, axis=1, keepdims=True)
    w_ref[...] = jnp.concatenate(
        [w, jnp.zeros((Bsz, 8 - topk), jnp.float32)], axis=1)


def _agg_kernel(idx_ref, w_ref, v_hbm, out_ref, buf, sems, topk: int,
                blk_l: int, L: int):
    b = pl.program_id(0)
    j = pl.program_id(1)
    l0 = j * blk_l
    n = blk_l + 8
    copies = []
    for i in range(topk):
        base = jax.lax.rem(l0 + idx_ref[i], L)
        off = jax.lax.rem(base, 8)
        aligned = pl.multiple_of(base - off, 8)
        c = pltpu.make_async_copy(v_hbm.at[b, pl.ds(aligned, n), :],
                                  buf.at[i], sems.at[i])
        c.start()
        copies.append((c, off))
    acc = None
    for i, (c, off) in enumerate(copies):
        c.wait()
        wv = w_ref[b, i]
        shift = jnp.where(off == 0, 0, n - off)
        rolled = pltpu.roll(buf[i], shift, axis=0)
        term = rolled[0:blk_l, :] * wv
        acc = term if acc is None else acc + term
    out_ref[0] = acc


def kernel(queries, keys, values, attn_mask):
    B, L, H, E = queries.shape
    HE = H * E
    topk = int(math.log(L))
    blk_l = 256
    q2 = queries.reshape(B, L, HE)
    k2 = keys.reshape(B, L, HE)
    v2 = values.reshape(B, L, HE)
    v_pad = jnp.concatenate([v2, v2[:, :blk_l + 8]], axis=1)

    J = L // _TILE_R
    s_tiles = pl.pallas_call(
        _corr_kernel,
        grid=(B, J),
        in_specs=[
            pl.BlockSpec((1, _TILE_R, HE), lambda b, j: (b, j, 0)),
            pl.BlockSpec((1, L, HE), lambda b, j: (b, 0, 0)),
        ],
        out_specs=pl.BlockSpec((1, 1, 1, L), lambda b, j: (b, j, 0, 0)),
        out_shape=jax.ShapeDtypeStruct((B, J, 1, L), jnp.float32),
    )(q2, k2)

    idx, w = pl.pallas_call(
        lambda c, i, wo: _topk_kernel(c, i, wo, topk),
        in_specs=[pl.BlockSpec((B, J, 1, L), lambda: (0, 0, 0, 0))],
        out_specs=[
            pl.BlockSpec(memory_space=pltpu.SMEM),
            pl.BlockSpec((B, 8), lambda: (0, 0)),
        ],
        out_shape=[
            jax.ShapeDtypeStruct((8,), jnp.int32),
            jax.ShapeDtypeStruct((B, 8), jnp.float32),
        ],
    )(s_tiles)

    out = pl.pallas_call(
        lambda i, wi, v, o, buf, sems: _agg_kernel(i, wi, v, o, buf, sems,
                                                   topk, blk_l, L),
        grid=(B, L // blk_l),
        in_specs=[
            pl.BlockSpec(memory_space=pltpu.SMEM),
            pl.BlockSpec(memory_space=pltpu.SMEM),
            pl.BlockSpec(memory_space=pl.ANY),
        ],
        out_specs=pl.BlockSpec((1, blk_l, HE), lambda b, j: (b, j, 0)),
        out_shape=jax.ShapeDtypeStruct((B, L, HE), jnp.float32),
        scratch_shapes=[
            pltpu.VMEM((topk, blk_l + 8, HE), jnp.float32),
            pltpu.SemaphoreType.DMA((topk,)),
        ],
    )(idx, w, v_pad)

    return out.reshape(B, L, H, E)
